# Initial kernel scaffold; baseline (speedup 1.0000x reference)
import functools

import jax
import jax.numpy as jnp
from jax import lax
from jax.experimental import pallas as pl
from jax.experimental.pallas import tpu as pltpu
from jax.experimental.pallas import tpu_sc as plsc

NC = 2
NS = 16
L = 16

N = 10000
D = 128
E = 320000

NPAD = 10240
ROWS_PER_TILE = NPAD // (NC * NS)
ROWS_PER_SC_TILE = NPAD // NS
EPAD = 323584
ECHUNKS = EPAD // (NC * NS * 128)
PCHUNKS = EPAD // (NS * 128)
DUMMY = NPAD - 1

_mesh = plsc.VectorSubcoreMesh(core_axis_name="c", subcore_axis_name="s")

_f32 = jnp.float32
_i32 = jnp.int32


def _rsqrt16(x):
    i = plsc.bitcast(x, _i32)
    i = jnp.int32(0x5F3759DF) - lax.shift_right_logical(i, 1)
    y = plsc.bitcast(i, _f32)
    for _ in range(3):
        y = y * (jnp.float32(1.5) - jnp.float32(0.5) * x * y * y)
    return y


def _bcast16(ref, idx_scalar):
    idx = jnp.zeros((L,), _i32) + idx_scalar
    return plsc.load_gather(ref, [idx])


def _prep_body(col_ref, feat_ref, dis_ref, dis2_ref, y0_ref,
               cnt_sh, colbuf, onesbuf, cntbuf, disb, dis2b, fbuf, sem):
    cid = lax.axis_index("c")
    sid = lax.axis_index("s")

    @pl.when(cid == 0)
    def _():
        for i in range(ROWS_PER_SC_TILE // L):
            cntbuf[pl.ds(i * L, L)] = jnp.zeros((L,), _f32)
        for i in range(128 // L):
            onesbuf[pl.ds(i * L, L)] = jnp.ones((L,), _f32)
        pltpu.sync_copy(cntbuf, cnt_sh.at[pl.ds(sid * ROWS_PER_SC_TILE,
                                                ROWS_PER_SC_TILE)])
        pltpu.sync_copy(col_ref.at[pl.ds(sid * PCHUNKS, PCHUNKS)], colbuf)
        plsc.subcore_barrier()

        def count_step(j, carry):
            pltpu.sync_copy(onesbuf, cnt_sh.at[colbuf.at[j]], add=True)
            return carry
        lax.fori_loop(0, PCHUNKS, count_step, 0)
        plsc.subcore_barrier()

        base = sid * ROWS_PER_SC_TILE
        pltpu.sync_copy(cnt_sh.at[pl.ds(base, ROWS_PER_SC_TILE)], cntbuf)
        for i in range(ROWS_PER_SC_TILE // L):
            q = pl.ds(i * L, L)
            deg = cntbuf[q] + jnp.float32(1.0)
            disb[q] = _rsqrt16(deg)
            dis2b[q] = jnp.float32(1.0) / deg
        pltpu.sync_copy(disb, dis_ref.at[pl.ds(base, ROWS_PER_SC_TILE)])
        pltpu.sync_copy(dis2b, dis2_ref.at[pl.ds(base, ROWS_PER_SC_TILE)])

        for b in range(ROWS_PER_SC_TILE // 128):
            off = (base + b * 128) * D
            pltpu.sync_copy(feat_ref.at[pl.ds(off, 128 * D)], fbuf)

            def scale_step(r, carry):
                disv = _bcast16(disb, b * 128 + r)
                for l in range(D // L):
                    q = pl.ds(r * D + l * L, L)
                    fbuf[q] = fbuf[q] * disv
                return carry
            lax.fori_loop(0, 128, scale_step, 0)
            pltpu.sync_copy(fbuf, y0_ref.at[pl.ds(off, 128 * D)])


_prep = pl.kernel(
    _prep_body,
    out_type=(
        jax.ShapeDtypeStruct((NPAD,), _f32),
        jax.ShapeDtypeStruct((NPAD,), _f32),
        jax.ShapeDtypeStruct((NPAD * D,), _f32),
    ),
    mesh=_mesh,
    scratch_types=[
        pltpu.VMEM_SHARED((NPAD,), _f32),
        pltpu.VMEM((PCHUNKS, 128), _i32),
        pltpu.VMEM((128,), _f32),
        pltpu.VMEM((ROWS_PER_SC_TILE,), _f32),
        pltpu.VMEM((ROWS_PER_SC_TILE,), _f32),
        pltpu.VMEM((ROWS_PER_SC_TILE,), _f32),
        pltpu.VMEM((128 * D,), _f32),
        pltpu.SemaphoreType.DMA,
    ],
)


def _scatter_body(y_ref, row_ref, col_ref, zeros_ref, p_ref,
                  acc, rowbuf, colbuf, gbuf, sem):
    cid = lax.axis_index("c")
    sid = lax.axis_index("s")
    wid = cid * NS + sid

    nbase = sid * ROWS_PER_SC_TILE
    pltpu.sync_copy(zeros_ref.at[pl.ds(nbase, ROWS_PER_SC_TILE)],
                    acc.at[pl.ds(nbase, ROWS_PER_SC_TILE)])
    pltpu.sync_copy(row_ref.at[pl.ds(wid * ECHUNKS, ECHUNKS)], rowbuf)
    pltpu.sync_copy(col_ref.at[pl.ds(wid * ECHUNKS, ECHUNKS)], colbuf)
    plsc.subcore_barrier()

    def edge_step(j, carry):
        pltpu.async_copy(y_ref.at[rowbuf.at[j]], gbuf, sem).wait()
        pltpu.sync_copy(gbuf, acc.at[colbuf.at[j]], add=True)
        return carry
    lax.fori_loop(0, ECHUNKS, edge_step, 0)
    plsc.subcore_barrier()

    pltpu.sync_copy(acc.at[pl.ds(nbase, ROWS_PER_SC_TILE)],
                    p_ref.at[cid, pl.ds(nbase, ROWS_PER_SC_TILE)])


_scatter = pl.kernel(
    _scatter_body,
    out_type=jax.ShapeDtypeStruct((NC, NPAD, D), _f32),
    mesh=_mesh,
    scratch_types=[
        pltpu.VMEM_SHARED((NPAD, D), _f32),
        pltpu.VMEM((ECHUNKS, 128), _i32),
        pltpu.VMEM((ECHUNKS, 128), _i32),
        pltpu.VMEM((128, D), _f32),
        pltpu.SemaphoreType.DMA,
    ],
)


_CROWS = 64
_CB = _CROWS * D


def _combine_body(p_ref, y_ref, dis_ref, dis2_ref, x_ref, yn_ref,
                  p0b, p1b, yb, xb, ynb, disb, dis2b, sem):
    cid = lax.axis_index("c")
    sid = lax.axis_index("s")
    wid = cid * NS + sid
    base = wid * ROWS_PER_TILE

    pltpu.sync_copy(dis_ref.at[pl.ds(base, ROWS_PER_TILE)], disb)
    pltpu.sync_copy(dis2_ref.at[pl.ds(base, ROWS_PER_TILE)], dis2b)

    for c in range(ROWS_PER_TILE // _CROWS):
        off = (base + c * _CROWS) * D
        pltpu.sync_copy(p_ref.at[pl.ds(off, _CB)], p0b)
        pltpu.sync_copy(p_ref.at[pl.ds(NPAD * D + off, _CB)], p1b)
        pltpu.sync_copy(y_ref.at[pl.ds(off, _CB)], yb)

        def row_step(r, carry):
            disv = _bcast16(disb, c * _CROWS + r)
            dis2v = _bcast16(dis2b, c * _CROWS + r)
            for l in range(D // L):
                q = pl.ds(r * D + l * L, L)
                s = p0b[q] + p1b[q] + yb[q]
                xb[q] = disv * s
                ynb[q] = dis2v * s
            return carry
        lax.fori_loop(0, _CROWS, row_step, 0)

        pltpu.sync_copy(xb, x_ref.at[pl.ds(off, _CB)])
        pltpu.sync_copy(ynb, yn_ref.at[pl.ds(off, _CB)])


_combine = pl.kernel(
    _combine_body,
    out_type=(
        jax.ShapeDtypeStruct((NPAD * D,), _f32),
        jax.ShapeDtypeStruct((NPAD * D,), _f32),
    ),
    mesh=_mesh,
    scratch_types=[
        pltpu.VMEM((_CB,), _f32),
        pltpu.VMEM((_CB,), _f32),
        pltpu.VMEM((_CB,), _f32),
        pltpu.VMEM((_CB,), _f32),
        pltpu.VMEM((_CB,), _f32),
        pltpu.VMEM((ROWS_PER_TILE,), _f32),
        pltpu.VMEM((ROWS_PER_TILE,), _f32),
        pltpu.SemaphoreType.DMA,
    ],
)


K_HOPS = 8


def kernel(feature, edge_index):
    row = edge_index[0]
    col = edge_index[1]
    pad = EPAD - E
    rowp = jnp.concatenate([row, jnp.zeros((pad,), _i32)]).reshape(EPAD // 128, 128)
    colp = jnp.concatenate([col, jnp.full((pad,), DUMMY, _i32)]).reshape(EPAD // 128, 128)
    featp = jnp.pad(feature, ((0, NPAD - N), (0, 0)))

    dis, dis2, y0f = _prep(colp, featp.reshape(-1))

    zeros = jnp.zeros((NPAD, D), _f32)
    y2d = y0f.reshape(NPAD, D)
    outs = [feature]
    for _ in range(K_HOPS):
        p = _scatter(y2d, rowp, colp, zeros)
        xf, ynf = _combine(p.reshape(-1), y2d.reshape(-1), dis, dis2)
        outs.append(xf.reshape(NPAD, D)[:N])
        y2d = ynf.reshape(NPAD, D)
    return jnp.concatenate(outs, axis=1)


# speedup vs baseline: 4.9563x; 4.9563x over previous
import functools

import jax
import jax.numpy as jnp
from jax import lax
from jax.experimental import pallas as pl
from jax.experimental.pallas import tpu as pltpu
from jax.experimental.pallas import tpu_sc as plsc

NC = 2
NS = 16
L = 16

N = 10000
D = 128
E = 320000

NPAD = 10240
ROWS_PER_TILE = NPAD // (NC * NS)
ROWS_PER_SC_TILE = NPAD // NS
EPAD = 327680
ECHUNKS = EPAD // (NC * NS * 128)
PCHUNKS = EPAD // (NS * 128)
DUMMY = NPAD - 1

_mesh = plsc.VectorSubcoreMesh(core_axis_name="c", subcore_axis_name="s")

_f32 = jnp.float32
_i32 = jnp.int32


def _rsqrt16(x):
    i = lax.bitcast_convert_type(x, _i32)
    i = jnp.int32(0x5F3759DF) - lax.shift_right_logical(i, 1)
    y = lax.bitcast_convert_type(i, _f32)
    for _ in range(3):
        y = y * (jnp.float32(1.5) - jnp.float32(0.5) * x * y * y)
    return y


def _bcast16(ref, idx_scalar):
    idx = jnp.zeros((L,), _i32) + idx_scalar
    return plsc.load_gather(ref, [idx])


def _prep_body(col_ref, feat_ref, dis_ref, dis2_ref, y0_ref,
               cnt_sh, colbuf, onesbuf, cntbuf, disb, dis2b, fbuf, sem):
    cid = lax.axis_index("c")
    sid = lax.axis_index("s")

    @pl.when(cid == 0)
    def _():
        for i in range(ROWS_PER_SC_TILE // L):
            cntbuf[pl.ds(i * L, L)] = jnp.zeros((L,), _f32)
        for i in range(128 // L):
            onesbuf[pl.ds(i * L, L)] = jnp.ones((L,), _f32)
        pltpu.sync_copy(cntbuf, cnt_sh.at[pl.ds(sid * ROWS_PER_SC_TILE,
                                                ROWS_PER_SC_TILE)])
        pltpu.sync_copy(col_ref.at[pl.ds(sid * PCHUNKS, PCHUNKS)], colbuf)
        plsc.subcore_barrier()

        def count_step(j, carry):
            pltpu.sync_copy(onesbuf, cnt_sh.at[colbuf.at[j]], add=True)
            return carry
        lax.fori_loop(0, PCHUNKS, count_step, 0)
        plsc.subcore_barrier()

        base = sid * ROWS_PER_SC_TILE
        pltpu.sync_copy(cnt_sh.at[pl.ds(base, ROWS_PER_SC_TILE)], cntbuf)
        for i in range(ROWS_PER_SC_TILE // L):
            q = pl.ds(i * L, L)
            deg = cntbuf[q] + jnp.float32(1.0)
            disb[q] = _rsqrt16(deg)
            dis2b[q] = jnp.float32(1.0) / deg
        pltpu.sync_copy(disb, dis_ref.at[pl.ds(base, ROWS_PER_SC_TILE)])
        pltpu.sync_copy(dis2b, dis2_ref.at[pl.ds(base, ROWS_PER_SC_TILE)])

        for b in range(ROWS_PER_SC_TILE // 128):
            off = (base + b * 128) * D
            pltpu.sync_copy(feat_ref.at[pl.ds(off, 128 * D)], fbuf)

            def scale_step(r, carry):
                disv = _bcast16(disb, b * 128 + r)
                for l in range(D // L):
                    q = pl.ds(r * D + l * L, L)
                    fbuf[q] = fbuf[q] * disv
                return carry
            lax.fori_loop(0, 128, scale_step, 0)
            pltpu.sync_copy(fbuf, y0_ref.at[pl.ds(off, 128 * D)])


_prep = pl.kernel(
    _prep_body,
    out_type=(
        jax.ShapeDtypeStruct((NPAD,), _f32),
        jax.ShapeDtypeStruct((NPAD,), _f32),
        jax.ShapeDtypeStruct((NPAD * D,), _f32),
    ),
    mesh=_mesh,
    compiler_params=pltpu.CompilerParams(needs_layout_passes=False),
    scratch_types=[
        pltpu.VMEM_SHARED((NPAD,), _f32),
        pltpu.VMEM((PCHUNKS, 128), _i32),
        pltpu.VMEM((128,), _f32),
        pltpu.VMEM((ROWS_PER_SC_TILE,), _f32),
        pltpu.VMEM((ROWS_PER_SC_TILE,), _f32),
        pltpu.VMEM((ROWS_PER_SC_TILE,), _f32),
        pltpu.VMEM((128 * D,), _f32),
        pltpu.SemaphoreType.DMA,
    ],
)


def _scatter_body(y_ref, row_ref, col_ref, zeros_ref, p_ref,
                  acc, rowbuf, colbuf, gbuf, sem):
    cid = lax.axis_index("c")
    sid = lax.axis_index("s")
    wid = cid * NS + sid

    nbase = sid * ROWS_PER_SC_TILE
    pltpu.sync_copy(zeros_ref.at[pl.ds(nbase, ROWS_PER_SC_TILE)],
                    acc.at[pl.ds(nbase, ROWS_PER_SC_TILE)])
    pltpu.sync_copy(row_ref.at[pl.ds(wid * ECHUNKS, ECHUNKS)], rowbuf)
    pltpu.sync_copy(col_ref.at[pl.ds(wid * ECHUNKS, ECHUNKS)], colbuf)
    plsc.subcore_barrier()

    def edge_step(j, carry):
        pltpu.async_copy(y_ref.at[rowbuf.at[j]], gbuf, sem).wait()
        pltpu.sync_copy(gbuf, acc.at[colbuf.at[j]], add=True)
        return carry
    lax.fori_loop(0, ECHUNKS, edge_step, 0)
    plsc.subcore_barrier()

    pltpu.sync_copy(acc.at[pl.ds(nbase, ROWS_PER_SC_TILE)],
                    p_ref.at[cid, pl.ds(nbase, ROWS_PER_SC_TILE)])


_scatter = pl.kernel(
    _scatter_body,
    out_type=jax.ShapeDtypeStruct((NC, NPAD, D), _f32),
    mesh=_mesh,
    compiler_params=pltpu.CompilerParams(needs_layout_passes=False),
    scratch_types=[
        pltpu.VMEM_SHARED((NPAD, D), _f32),
        pltpu.VMEM((ECHUNKS, 128), _i32),
        pltpu.VMEM((ECHUNKS, 128), _i32),
        pltpu.VMEM((128, D), _f32),
        pltpu.SemaphoreType.DMA,
    ],
)


_CROWS = 64
_CB = _CROWS * D


def _combine_body(p_ref, y_ref, dis_ref, dis2_ref, x_ref, yn_ref,
                  p0b, p1b, yb, xb, ynb, disb, dis2b, sem):
    cid = lax.axis_index("c")
    sid = lax.axis_index("s")
    wid = cid * NS + sid
    base = wid * ROWS_PER_TILE

    pltpu.sync_copy(dis_ref.at[pl.ds(base, ROWS_PER_TILE)], disb)
    pltpu.sync_copy(dis2_ref.at[pl.ds(base, ROWS_PER_TILE)], dis2b)

    for c in range(ROWS_PER_TILE // _CROWS):
        off = (base + c * _CROWS) * D
        pltpu.sync_copy(p_ref.at[pl.ds(off, _CB)], p0b)
        pltpu.sync_copy(p_ref.at[pl.ds(NPAD * D + off, _CB)], p1b)
        pltpu.sync_copy(y_ref.at[pl.ds(off, _CB)], yb)

        def row_step(r, carry):
            disv = _bcast16(disb, c * _CROWS + r)
            dis2v = _bcast16(dis2b, c * _CROWS + r)
            for l in range(D // L):
                q = pl.ds(r * D + l * L, L)
                s = p0b[q] + p1b[q] + yb[q]
                xb[q] = disv * s
                ynb[q] = dis2v * s
            return carry
        lax.fori_loop(0, _CROWS, row_step, 0)

        pltpu.sync_copy(xb, x_ref.at[pl.ds(off, _CB)])
        pltpu.sync_copy(ynb, yn_ref.at[pl.ds(off, _CB)])


_combine = pl.kernel(
    _combine_body,
    out_type=(
        jax.ShapeDtypeStruct((NPAD * D,), _f32),
        jax.ShapeDtypeStruct((NPAD * D,), _f32),
    ),
    mesh=_mesh,
    compiler_params=pltpu.CompilerParams(needs_layout_passes=False),
    scratch_types=[
        pltpu.VMEM((_CB,), _f32),
        pltpu.VMEM((_CB,), _f32),
        pltpu.VMEM((_CB,), _f32),
        pltpu.VMEM((_CB,), _f32),
        pltpu.VMEM((_CB,), _f32),
        pltpu.VMEM((ROWS_PER_TILE,), _f32),
        pltpu.VMEM((ROWS_PER_TILE,), _f32),
        pltpu.SemaphoreType.DMA,
    ],
)


K_HOPS = 8


def kernel(feature, edge_index):
    row = edge_index[0]
    col = edge_index[1]
    pad = EPAD - E
    rowp = jnp.concatenate([row, jnp.zeros((pad,), _i32)]).reshape(EPAD // 128, 128)
    colp = jnp.concatenate([col, jnp.full((pad,), DUMMY, _i32)]).reshape(EPAD // 128, 128)
    featp = jnp.pad(feature, ((0, NPAD - N), (0, 0)))

    dis, dis2, y0f = _prep(colp, featp.reshape(-1))

    zeros = jnp.zeros((NPAD, D), _f32)
    y2d = y0f.reshape(NPAD, D)
    outs = [feature]
    for _ in range(K_HOPS):
        p = _scatter(y2d, rowp, colp, zeros)
        xf, ynf = _combine(p.reshape(-1), y2d.reshape(-1), dis, dis2)
        outs.append(xf.reshape(NPAD, D)[:N])
        y2d = ynf.reshape(NPAD, D)
    return jnp.concatenate(outs, axis=1)


# speedup vs baseline: 16.5168x; 3.3325x over previous
import functools

import jax
import jax.numpy as jnp
from jax import lax
from jax.experimental import pallas as pl
from jax.experimental.pallas import tpu as pltpu
from jax.experimental.pallas import tpu_sc as plsc

NC = 2
NS = 16
L = 16

N = 10000
D = 128
E = 320000

NPAD = 10240
ROWS_PER_TILE = NPAD // (NC * NS)
ROWS_PER_SC_TILE = NPAD // NS
EPAD = 327680
ECHUNKS = EPAD // (NC * NS * 128)
PCHUNKS = EPAD // (NS * 128)
DUMMY = NPAD - 1

_mesh = plsc.VectorSubcoreMesh(core_axis_name="c", subcore_axis_name="s")

_f32 = jnp.float32
_i32 = jnp.int32


def _rsqrt16(x):
    i = lax.bitcast_convert_type(x, _i32)
    i = jnp.int32(0x5F3759DF) - lax.shift_right_logical(i, 1)
    y = lax.bitcast_convert_type(i, _f32)
    for _ in range(3):
        y = y * (jnp.float32(1.5) - jnp.float32(0.5) * x * y * y)
    return y


def _bcast16(ref, idx_scalar):
    idx = jnp.zeros((L,), _i32) + idx_scalar
    return plsc.load_gather(ref, [idx])


def _prep_body(col_ref, feat_ref, dis_ref, dis2_ref, y0_ref,
               cnt_sh, colbuf, onesbuf, cntbuf, disb, dis2b, fbuf, sem):
    cid = lax.axis_index("c")
    sid = lax.axis_index("s")

    @pl.when(cid == 0)
    def _():
        for i in range(ROWS_PER_SC_TILE // L):
            cntbuf[pl.ds(i * L, L)] = jnp.zeros((L,), _f32)
        for i in range(128 // L):
            onesbuf[pl.ds(i * L, L)] = jnp.ones((L,), _f32)
        pltpu.sync_copy(cntbuf, cnt_sh.at[pl.ds(sid * ROWS_PER_SC_TILE,
                                                ROWS_PER_SC_TILE)])
        pltpu.sync_copy(col_ref.at[pl.ds(sid * PCHUNKS, PCHUNKS)], colbuf)
        plsc.subcore_barrier()

        def count_step(j, carry):
            pltpu.sync_copy(onesbuf, cnt_sh.at[colbuf.at[j]], add=True)
            return carry
        lax.fori_loop(0, PCHUNKS, count_step, 0)
        plsc.subcore_barrier()

        base = sid * ROWS_PER_SC_TILE
        pltpu.sync_copy(cnt_sh.at[pl.ds(base, ROWS_PER_SC_TILE)], cntbuf)
        for i in range(ROWS_PER_SC_TILE // L):
            q = pl.ds(i * L, L)
            deg = cntbuf[q] + jnp.float32(1.0)
            disb[q] = _rsqrt16(deg)
            dis2b[q] = jnp.float32(1.0) / deg
        pltpu.sync_copy(disb, dis_ref.at[pl.ds(base, ROWS_PER_SC_TILE)])
        pltpu.sync_copy(dis2b, dis2_ref.at[pl.ds(base, ROWS_PER_SC_TILE)])

        for b in range(ROWS_PER_SC_TILE // 128):
            off = (base + b * 128) * D
            pltpu.sync_copy(feat_ref.at[pl.ds(off, 128 * D)], fbuf)

            def scale_step(r, carry):
                disv = _bcast16(disb, b * 128 + r)
                for l in range(D // L):
                    q = pl.ds(r * D + l * L, L)
                    fbuf[q] = fbuf[q] * disv
                return carry
            lax.fori_loop(0, 128, scale_step, 0)
            pltpu.sync_copy(fbuf, y0_ref.at[pl.ds(off, 128 * D)])


_prep = pl.kernel(
    _prep_body,
    out_type=(
        jax.ShapeDtypeStruct((NPAD,), _f32),
        jax.ShapeDtypeStruct((NPAD,), _f32),
        jax.ShapeDtypeStruct((NPAD * D,), _f32),
    ),
    mesh=_mesh,
    compiler_params=pltpu.CompilerParams(needs_layout_passes=False),
    scratch_types=[
        pltpu.VMEM_SHARED((NPAD,), _f32),
        pltpu.VMEM((PCHUNKS, 128), _i32),
        pltpu.VMEM((128,), _f32),
        pltpu.VMEM((ROWS_PER_SC_TILE,), _f32),
        pltpu.VMEM((ROWS_PER_SC_TILE,), _f32),
        pltpu.VMEM((ROWS_PER_SC_TILE,), _f32),
        pltpu.VMEM((128 * D,), _f32),
        pltpu.SemaphoreType.DMA,
    ],
)


_NB = 2


def _scatter_body(y_ref, packed_ref, zeros_ref, p_ref,
                  acc, packedbuf, rc0, rc1, cc0, cc1, gb0, gb1, s0, s1):
    cid = lax.axis_index("c")
    sid = lax.axis_index("s")
    wid = cid * NS + sid
    bufs = (gb0, gb1)
    rcs = (rc0, rc1)
    ccs = (cc0, cc1)
    sems = (s0, s1)

    nbase = sid * ROWS_PER_SC_TILE
    pltpu.sync_copy(zeros_ref.at[pl.ds(nbase, ROWS_PER_SC_TILE)],
                    acc.at[pl.ds(nbase, ROWS_PER_SC_TILE)])
    pltpu.sync_copy(packed_ref.at[pl.ds(wid * ECHUNKS * 128, ECHUNKS * 128)],
                    packedbuf)
    plsc.subcore_barrier()

    def unpack(i, rb, cb):
        for l in range(128 // L):
            pk = packedbuf[pl.ds(i * 128 + l * L, L)]
            rb[pl.ds(l * L, L)] = pk & jnp.int32(0xFFFF)
            cb[pl.ds(l * L, L)] = lax.shift_right_logical(pk, 16)

    for b in range(_NB):
        unpack(b, rcs[b], ccs[b])
        pltpu.async_copy(y_ref.at[rcs[b]], bufs[b], sems[b])

    def block(k, carry):
        for b in range(_NB):
            i = k * _NB + b
            pltpu.make_async_copy(y_ref.at[rcs[b]], bufs[b], sems[b]).wait()
            pltpu.sync_copy(bufs[b], acc.at[ccs[b]], add=True)
            nxt = i + _NB

            @pl.when(nxt < ECHUNKS)
            def _():
                unpack(nxt, rcs[b], ccs[b])
                pltpu.async_copy(y_ref.at[rcs[b]], bufs[b], sems[b])
        return carry
    lax.fori_loop(0, ECHUNKS // _NB, block, 0)
    plsc.subcore_barrier()

    pltpu.sync_copy(acc.at[pl.ds(nbase, ROWS_PER_SC_TILE)],
                    p_ref.at[cid, pl.ds(nbase, ROWS_PER_SC_TILE)])


_scatter = pl.kernel(
    _scatter_body,
    out_type=jax.ShapeDtypeStruct((NC, NPAD, D), _f32),
    mesh=_mesh,
    compiler_params=pltpu.CompilerParams(needs_layout_passes=False),
    scratch_types=[
        pltpu.VMEM_SHARED((NPAD, D), _f32),
        pltpu.VMEM((ECHUNKS * 128,), _i32),
        pltpu.VMEM((128,), _i32),
        pltpu.VMEM((128,), _i32),
        pltpu.VMEM((128,), _i32),
        pltpu.VMEM((128,), _i32),
        pltpu.VMEM((128, D), _f32),
        pltpu.VMEM((128, D), _f32),
        pltpu.SemaphoreType.DMA,
        pltpu.SemaphoreType.DMA,
    ],
)


_CROWS = 64
_CB = _CROWS * D


def _combine_body(p_ref, y_ref, dis_ref, dis2_ref, x_ref, yn_ref,
                  p0b, p1b, yb, xb, ynb, disb, dis2b, sem):
    cid = lax.axis_index("c")
    sid = lax.axis_index("s")
    wid = cid * NS + sid
    base = wid * ROWS_PER_TILE

    pltpu.sync_copy(dis_ref.at[pl.ds(base, ROWS_PER_TILE)], disb)
    pltpu.sync_copy(dis2_ref.at[pl.ds(base, ROWS_PER_TILE)], dis2b)

    for c in range(ROWS_PER_TILE // _CROWS):
        off = (base + c * _CROWS) * D
        pltpu.sync_copy(p_ref.at[pl.ds(off, _CB)], p0b)
        pltpu.sync_copy(p_ref.at[pl.ds(NPAD * D + off, _CB)], p1b)
        pltpu.sync_copy(y_ref.at[pl.ds(off, _CB)], yb)

        def row_step(r, carry):
            disv = _bcast16(disb, c * _CROWS + r)
            dis2v = _bcast16(dis2b, c * _CROWS + r)
            for l in range(D // L):
                q = pl.ds(r * D + l * L, L)
                s = p0b[q] + p1b[q] + yb[q]
                xb[q] = disv * s
                ynb[q] = dis2v * s
            return carry
        lax.fori_loop(0, _CROWS, row_step, 0)

        pltpu.sync_copy(xb, x_ref.at[pl.ds(off, _CB)])
        pltpu.sync_copy(ynb, yn_ref.at[pl.ds(off, _CB)])


_combine = pl.kernel(
    _combine_body,
    out_type=(
        jax.ShapeDtypeStruct((NPAD * D,), _f32),
        jax.ShapeDtypeStruct((NPAD * D,), _f32),
    ),
    mesh=_mesh,
    compiler_params=pltpu.CompilerParams(needs_layout_passes=False),
    scratch_types=[
        pltpu.VMEM((_CB,), _f32),
        pltpu.VMEM((_CB,), _f32),
        pltpu.VMEM((_CB,), _f32),
        pltpu.VMEM((_CB,), _f32),
        pltpu.VMEM((_CB,), _f32),
        pltpu.VMEM((ROWS_PER_TILE,), _f32),
        pltpu.VMEM((ROWS_PER_TILE,), _f32),
        pltpu.SemaphoreType.DMA,
    ],
)


K_HOPS = 8


def kernel(feature, edge_index):
    row = edge_index[0]
    col = edge_index[1]
    pad = EPAD - E
    padv = N + jnp.arange(pad, dtype=_i32) % (NPAD - N)
    rowf = jnp.concatenate([row, padv])
    colf = jnp.concatenate([col, padv])
    colp = colf.reshape(EPAD // 128, 128)
    packed = rowf | (colf << 16)
    featp = jnp.pad(feature, ((0, NPAD - N), (0, 0)))

    dis, dis2, y0f = _prep(colp, featp.reshape(-1))

    zeros = jnp.zeros((NPAD, D), _f32)
    y2d = y0f.reshape(NPAD, D)
    outs = [feature]
    for _ in range(K_HOPS):
        p = _scatter(y2d, packed, zeros)
        xf, ynf = _combine(p.reshape(-1), y2d.reshape(-1), dis, dis2)
        outs.append(xf.reshape(NPAD, D)[:N])
        y2d = ynf.reshape(NPAD, D)
    return jnp.concatenate(outs, axis=1)


# speedup vs baseline: 20.5129x; 1.2419x over previous
import functools

import jax
import jax.numpy as jnp
from jax import lax
from jax.experimental import pallas as pl
from jax.experimental.pallas import tpu as pltpu
from jax.experimental.pallas import tpu_sc as plsc

NC = 2
NS = 16
L = 16

N = 10000
D = 128
E = 320000

NPAD = 10240
ROWS_PER_TILE = NPAD // (NC * NS)
ROWS_PER_SC_TILE = NPAD // NS
EPAD = 327680
ECHUNKS = EPAD // (NC * NS * 128)
PCHUNKS = EPAD // (NS * 128)
DUMMY = NPAD - 1

_mesh = plsc.VectorSubcoreMesh(core_axis_name="c", subcore_axis_name="s")

_f32 = jnp.float32
_i32 = jnp.int32


def _rsqrt16(x):
    i = lax.bitcast_convert_type(x, _i32)
    i = jnp.int32(0x5F3759DF) - lax.shift_right_logical(i, 1)
    y = lax.bitcast_convert_type(i, _f32)
    for _ in range(3):
        y = y * (jnp.float32(1.5) - jnp.float32(0.5) * x * y * y)
    return y


def _bcast16(ref, idx_scalar):
    idx = jnp.zeros((L,), _i32) + idx_scalar
    return plsc.load_gather(ref, [idx])


def _prep_body(col_ref, cnt_ref, cnt_sh, colbuf, onesbuf, cntbuf, sem):
    cid = lax.axis_index("c")
    sid = lax.axis_index("s")

    @pl.when(cid == 0)
    def _():
        for i in range(ROWS_PER_SC_TILE // L):
            cntbuf[pl.ds(i * L, L)] = jnp.zeros((L,), _f32)
        for i in range(128 // L):
            onesbuf[pl.ds(i * L, L)] = jnp.ones((L,), _f32)
        pltpu.sync_copy(cntbuf, cnt_sh.at[pl.ds(sid * ROWS_PER_SC_TILE,
                                                ROWS_PER_SC_TILE)])
        pltpu.sync_copy(col_ref.at[pl.ds(sid * PCHUNKS, PCHUNKS)], colbuf)
        plsc.subcore_barrier()

        def count_step(j, carry):
            pltpu.sync_copy(onesbuf, cnt_sh.at[colbuf.at[j]], add=True)
            return carry
        lax.fori_loop(0, PCHUNKS, count_step, 0)
        plsc.subcore_barrier()

        base = sid * ROWS_PER_SC_TILE
        pltpu.sync_copy(cnt_sh.at[pl.ds(base, ROWS_PER_SC_TILE)],
                        cnt_ref.at[pl.ds(base, ROWS_PER_SC_TILE)])


_prep = pl.kernel(
    _prep_body,
    out_type=jax.ShapeDtypeStruct((NPAD,), _f32),
    mesh=_mesh,
    compiler_params=pltpu.CompilerParams(needs_layout_passes=False),
    scratch_types=[
        pltpu.VMEM_SHARED((NPAD,), _f32),
        pltpu.VMEM((PCHUNKS, 128), _i32),
        pltpu.VMEM((128,), _f32),
        pltpu.VMEM((ROWS_PER_SC_TILE,), _f32),
        pltpu.SemaphoreType.DMA,
    ],
)


_NB = 2


def _scatter_body(y_ref, packed_ref, zeros_ref, p_ref,
                  acc, packedbuf, rc0, rc1, cc0, cc1, gb0, gb1, s0, s1):
    cid = lax.axis_index("c")
    sid = lax.axis_index("s")
    wid = cid * NS + sid
    bufs = (gb0, gb1)
    rcs = (rc0, rc1)
    ccs = (cc0, cc1)
    sems = (s0, s1)

    nbase = sid * ROWS_PER_SC_TILE
    pltpu.sync_copy(zeros_ref.at[pl.ds(nbase, ROWS_PER_SC_TILE)],
                    acc.at[pl.ds(nbase, ROWS_PER_SC_TILE)])
    pltpu.sync_copy(packed_ref.at[pl.ds(wid * ECHUNKS * 128, ECHUNKS * 128)],
                    packedbuf)
    plsc.subcore_barrier()

    def unpack(i, rb, cb):
        for l in range(128 // L):
            pk = packedbuf[pl.ds(i * 128 + l * L, L)]
            rb[pl.ds(l * L, L)] = pk & jnp.int32(0xFFFF)
            cb[pl.ds(l * L, L)] = lax.shift_right_logical(pk, 16)

    for b in range(_NB):
        unpack(b, rcs[b], ccs[b])
        pltpu.async_copy(y_ref.at[rcs[b]], bufs[b], sems[b])

    def block(k, carry):
        for b in range(_NB):
            i = k * _NB + b
            pltpu.make_async_copy(y_ref.at[rcs[b]], bufs[b], sems[b]).wait()
            pltpu.sync_copy(bufs[b], acc.at[ccs[b]], add=True)
            nxt = i + _NB

            @pl.when(nxt < ECHUNKS)
            def _():
                unpack(nxt, rcs[b], ccs[b])
                pltpu.async_copy(y_ref.at[rcs[b]], bufs[b], sems[b])
        return carry
    lax.fori_loop(0, ECHUNKS // _NB, block, 0)
    plsc.subcore_barrier()

    pltpu.sync_copy(acc.at[pl.ds(nbase, ROWS_PER_SC_TILE)],
                    p_ref.at[cid, pl.ds(nbase, ROWS_PER_SC_TILE)])


_scatter = pl.kernel(
    _scatter_body,
    out_type=jax.ShapeDtypeStruct((NC, NPAD, D), _f32),
    mesh=_mesh,
    compiler_params=pltpu.CompilerParams(needs_layout_passes=False),
    scratch_types=[
        pltpu.VMEM_SHARED((NPAD, D), _f32),
        pltpu.VMEM((ECHUNKS * 128,), _i32),
        pltpu.VMEM((128,), _i32),
        pltpu.VMEM((128,), _i32),
        pltpu.VMEM((128,), _i32),
        pltpu.VMEM((128,), _i32),
        pltpu.VMEM((128, D), _f32),
        pltpu.VMEM((128, D), _f32),
        pltpu.SemaphoreType.DMA,
        pltpu.SemaphoreType.DMA,
    ],
)


_TCR = 1024


def _prep_tc_body(cnt_ref, feat_ref, dis_ref, dis2_ref, y0_ref):
    deg = cnt_ref[...] + 1.0
    dis = lax.rsqrt(deg)
    dis_ref[...] = dis
    dis2_ref[...] = 1.0 / deg
    y0_ref[...] = dis * feat_ref[...]


_prep_tc = pl.pallas_call(
    _prep_tc_body,
    grid=(NPAD // _TCR,),
    in_specs=[
        pl.BlockSpec((_TCR, 1), lambda i: (i, 0)),
        pl.BlockSpec((_TCR, D), lambda i: (i, 0)),
    ],
    out_specs=[
        pl.BlockSpec((_TCR, 1), lambda i: (i, 0)),
        pl.BlockSpec((_TCR, 1), lambda i: (i, 0)),
        pl.BlockSpec((_TCR, D), lambda i: (i, 0)),
    ],
    out_shape=[
        jax.ShapeDtypeStruct((NPAD, 1), _f32),
        jax.ShapeDtypeStruct((NPAD, 1), _f32),
        jax.ShapeDtypeStruct((NPAD, D), _f32),
    ],
)


def _combine_tc_body(p_ref, y_ref, dis_ref, dis2_ref, x_ref, yn_ref):
    s = p_ref[0] + p_ref[1] + y_ref[...]
    x_ref[...] = dis_ref[...] * s
    yn_ref[...] = dis2_ref[...] * s


_combine_tc = pl.pallas_call(
    _combine_tc_body,
    grid=(NPAD // _TCR,),
    in_specs=[
        pl.BlockSpec((2, _TCR, D), lambda i: (0, i, 0)),
        pl.BlockSpec((_TCR, D), lambda i: (i, 0)),
        pl.BlockSpec((_TCR, 1), lambda i: (i, 0)),
        pl.BlockSpec((_TCR, 1), lambda i: (i, 0)),
    ],
    out_specs=[
        pl.BlockSpec((_TCR, D), lambda i: (i, 0)),
        pl.BlockSpec((_TCR, D), lambda i: (i, 0)),
    ],
    out_shape=[
        jax.ShapeDtypeStruct((NPAD, D), _f32),
        jax.ShapeDtypeStruct((NPAD, D), _f32),
    ],
)


K_HOPS = 8


def kernel(feature, edge_index):
    row = edge_index[0]
    col = edge_index[1]
    pad = EPAD - E
    padv = N + jnp.arange(pad, dtype=_i32) % (NPAD - N)
    rowf = jnp.concatenate([row, padv])
    colf = jnp.concatenate([col, padv])
    colp = colf.reshape(EPAD // 128, 128)
    packed = rowf | (colf << 16)
    featp = jnp.pad(feature, ((0, NPAD - N), (0, 0)))

    cnt = _prep(colp)
    dis, dis2, y2d = _prep_tc(cnt.reshape(NPAD, 1), featp)

    zeros = jnp.zeros((NPAD, D), _f32)
    outs = [feature]
    for _ in range(K_HOPS):
        p = _scatter(y2d, packed, zeros)
        x, y2d = _combine_tc(p, y2d, dis, dis2)
        outs.append(x[:N])
    return jnp.concatenate(outs, axis=1)
